# baseline (speedup 1.0000x reference)
import functools

import jax
import jax.numpy as jnp
from jax import lax
from jax.experimental import pallas as pl
from jax.experimental.pallas import tpu as pltpu
from jax.experimental.pallas import tpu_sc as plsc

N_CODES = 8192
E_DIM = 256
N_ROWS = 8192
TILE_R = 256
N_TILES = N_ROWS // TILE_R
IDX_CHUNK = 128


def _dist_argmin_body(x_ref, xn_ref, w_ref, wn_ref, idx_ref, loss_ref,
                      iotaf_ref):
    i = pl.program_id(0)
    x = x_ref[...]
    w = w_ref[...]

    @pl.when(i == 0)
    def _():
        iotaf_ref[...] = lax.broadcasted_iota(
            jnp.int32, (1, N_CODES), 1).astype(jnp.float32)

    mm = lax.dot_general(x, w, (((1,), (1,)), ((), ())),
                         preferred_element_type=jnp.float32)
    xn = xn_ref[...]
    wn = wn_ref[...]
    d = (xn + wn) - 2.0 * mm
    m = jnp.min(d, axis=1, keepdims=True)
    idxf = jnp.min(
        jnp.where(d == m, iotaf_ref[...], jnp.float32(N_CODES)), axis=1)
    idx_ref[0, 0, :] = idxf.astype(jnp.int32)
    mind = m[:, 0]

    @pl.when(i == 0)
    def _():
        loss_ref[...] = jnp.zeros((1, 1), jnp.float32)

    loss_ref[...] += jnp.sum(mind).reshape(1, 1)


def _dist_argmin(x2, xn, W, wn):
    n_tiles = x2.shape[0] // TILE_R
    return pl.pallas_call(
        _dist_argmin_body,
        grid=(n_tiles,),
        in_specs=[
            pl.BlockSpec((TILE_R, E_DIM), lambda i: (i, 0)),
            pl.BlockSpec((TILE_R, 1), lambda i: (i, 0)),
            pl.BlockSpec((N_CODES, E_DIM), lambda i: (0, 0)),
            pl.BlockSpec((1, N_CODES), lambda i: (0, 0)),
        ],
        out_specs=[
            pl.BlockSpec((1, 1, TILE_R), lambda i: (i, 0, 0)),
            pl.BlockSpec((1, 1), lambda i: (0, 0)),
        ],
        out_shape=[
            jax.ShapeDtypeStruct((n_tiles, 1, TILE_R), jnp.int32),
            jax.ShapeDtypeStruct((1, 1), jnp.float32),
        ],
        scratch_shapes=[
            pltpu.VMEM((1, N_CODES), jnp.float32),
        ],
    )(x2, xn, W, wn)


def _make_sc_gather(n_rows):
    info = plsc.get_sparse_core_info()
    nc, ns = info.num_cores, info.num_subcores
    nw = nc * ns
    b_per_w = n_rows // nw
    n_chunks = b_per_w // IDX_CHUNK
    mesh = plsc.VectorSubcoreMesh(core_axis_name="c", subcore_axis_name="s")

    @functools.partial(
        pl.kernel,
        mesh=mesh,
        out_type=jax.ShapeDtypeStruct((n_rows, E_DIM), jnp.float32),
        scratch_types=[
            pltpu.VMEM((n_chunks, IDX_CHUNK), jnp.int32),
            pltpu.VMEM((b_per_w, E_DIM), jnp.float32),
            pltpu.SemaphoreType.DMA,
        ],
    )
    def gather(table_hbm, idx_hbm, out_hbm, idx_v, rows_v, sem):
        wid = lax.axis_index("s") * nc + lax.axis_index("c")
        pltpu.sync_copy(idx_hbm.at[pl.ds(wid * n_chunks, n_chunks)], idx_v)
        for j in range(n_chunks):
            pltpu.async_copy(
                table_hbm.at[idx_v.at[j]],
                rows_v.at[pl.ds(j * IDX_CHUNK, IDX_CHUNK)],
                sem,
            ).wait()
        pltpu.sync_copy(rows_v, out_hbm.at[pl.ds(wid * b_per_w, b_per_w)])

    return gather


def kernel(inputs, W):
    x2 = jnp.transpose(inputs, (0, 2, 3, 4, 1)).reshape(-1, E_DIM)
    xn = jnp.sum(x2 ** 2, axis=1, keepdims=True)
    wn = jnp.sum(W ** 2, axis=1).reshape(1, N_CODES)
    idx3, loss_acc = _dist_argmin(x2, xn, W, wn)
    idx2d = idx3.reshape(N_ROWS // IDX_CHUNK, IDX_CHUNK)
    q = _make_sc_gather(N_ROWS)(W, idx2d)
    loss = loss_acc[0, 0] * (1.25 / (N_ROWS * E_DIM))
    out = jnp.transpose(q.reshape(2, 16, 16, 16, E_DIM), (0, 4, 1, 2, 3))
    return (loss, out)


# speedup vs baseline: 1.2348x; 1.2348x over previous
import functools

import jax
import jax.numpy as jnp
from jax import lax
from jax.experimental import pallas as pl
from jax.experimental.pallas import tpu as pltpu
from jax.experimental.pallas import tpu_sc as plsc

N_CODES = 8192
E_DIM = 256
N_ROWS = 8192
TILE_R = 1024
N_TILES = N_ROWS // TILE_R
IDX_CHUNK = 128


def _dist_argmin_body(x_ref, xn_ref, w_ref, wn_ref, idx_ref, loss_ref,
                      iotaf_ref):
    i = pl.program_id(0)
    x = x_ref[...]
    w = w_ref[...]

    @pl.when(i == 0)
    def _():
        iotaf_ref[...] = lax.broadcasted_iota(
            jnp.int32, (1, N_CODES), 1).astype(jnp.float32)

    mm = lax.dot_general(x, w, (((1,), (1,)), ((), ())),
                         preferred_element_type=jnp.float32)
    xn = xn_ref[...]
    wn = wn_ref[...]
    d = (xn + wn) - 2.0 * mm
    m = jnp.min(d, axis=1, keepdims=True)
    idxf = jnp.min(
        jnp.where(d == m, iotaf_ref[...], jnp.float32(N_CODES)), axis=1)
    idx_ref[0, 0, :] = idxf.astype(jnp.int32)
    mind = m[:, 0]

    @pl.when(i == 0)
    def _():
        loss_ref[...] = jnp.zeros((1, 1), jnp.float32)

    loss_ref[...] += jnp.sum(mind).reshape(1, 1)


def _dist_argmin(x2, xn, W, wn):
    n_tiles = x2.shape[0] // TILE_R
    return pl.pallas_call(
        _dist_argmin_body,
        grid=(n_tiles,),
        in_specs=[
            pl.BlockSpec((TILE_R, E_DIM), lambda i: (i, 0)),
            pl.BlockSpec((TILE_R, 1), lambda i: (i, 0)),
            pl.BlockSpec((N_CODES, E_DIM), lambda i: (0, 0)),
            pl.BlockSpec((1, N_CODES), lambda i: (0, 0)),
        ],
        out_specs=[
            pl.BlockSpec((1, 1, TILE_R), lambda i: (i, 0, 0)),
            pl.BlockSpec((1, 1), lambda i: (0, 0)),
        ],
        out_shape=[
            jax.ShapeDtypeStruct((n_tiles, 1, TILE_R), jnp.int32),
            jax.ShapeDtypeStruct((1, 1), jnp.float32),
        ],
        scratch_shapes=[
            pltpu.VMEM((1, N_CODES), jnp.float32),
        ],
    )(x2, xn, W, wn)


def _make_sc_gather(n_rows):
    info = plsc.get_sparse_core_info()
    nc, ns = info.num_cores, info.num_subcores
    nw = nc * ns
    b_per_w = n_rows // nw
    n_chunks = b_per_w // IDX_CHUNK
    mesh = plsc.VectorSubcoreMesh(core_axis_name="c", subcore_axis_name="s")

    @functools.partial(
        pl.kernel,
        mesh=mesh,
        out_type=jax.ShapeDtypeStruct((n_rows, E_DIM), jnp.float32),
        scratch_types=[
            pltpu.VMEM((n_chunks, IDX_CHUNK), jnp.int32),
            pltpu.VMEM((b_per_w, E_DIM), jnp.float32),
            pltpu.SemaphoreType.DMA,
        ],
    )
    def gather(table_hbm, idx_hbm, out_hbm, idx_v, rows_v, sem):
        wid = lax.axis_index("s") * nc + lax.axis_index("c")
        pltpu.sync_copy(idx_hbm.at[pl.ds(wid * n_chunks, n_chunks)], idx_v)
        for j in range(n_chunks):
            pltpu.async_copy(
                table_hbm.at[idx_v.at[j]],
                rows_v.at[pl.ds(j * IDX_CHUNK, IDX_CHUNK)],
                sem,
            ).wait()
        pltpu.sync_copy(rows_v, out_hbm.at[pl.ds(wid * b_per_w, b_per_w)])

    return gather


def kernel(inputs, W):
    x2 = jnp.transpose(inputs, (0, 2, 3, 4, 1)).reshape(-1, E_DIM)
    xn = jnp.sum(x2 ** 2, axis=1, keepdims=True)
    wn = jnp.sum(W ** 2, axis=1).reshape(1, N_CODES)
    idx3, loss_acc = _dist_argmin(x2, xn, W, wn)
    idx2d = idx3.reshape(N_ROWS // IDX_CHUNK, IDX_CHUNK)
    q = _make_sc_gather(N_ROWS)(W, idx2d)
    loss = loss_acc[0, 0] * (1.25 / (N_ROWS * E_DIM))
    out = jnp.transpose(q.reshape(2, 16, 16, 16, E_DIM), (0, 4, 1, 2, 3))
    return (loss, out)


# speedup vs baseline: 1.3048x; 1.0567x over previous
import functools

import jax
import jax.numpy as jnp
from jax import lax
from jax.experimental import pallas as pl
from jax.experimental.pallas import tpu as pltpu
from jax.experimental.pallas import tpu_sc as plsc

N_CODES = 8192
E_DIM = 256
N_ROWS = 8192
TILE_R = 1024
N_TILES = N_ROWS // TILE_R
IDX_CHUNK = 128


def _dist_argmin_body(x_ref, w_ref, wn_ref, idx_ref, loss_ref,
                      iotaf_ref):
    i = pl.program_id(0)
    x = x_ref[...]
    w = w_ref[...]

    @pl.when(i == 0)
    def _():
        iotaf_ref[...] = lax.broadcasted_iota(
            jnp.int32, (1, N_CODES), 1).astype(jnp.float32)

    mm = lax.dot_general(x, w, (((1,), (1,)), ((), ())),
                         preferred_element_type=jnp.float32)
    xn = jnp.sum(x * x, axis=1, keepdims=True)
    wn = wn_ref[...]
    d = (xn + wn) - 2.0 * mm
    m = jnp.min(d, axis=1, keepdims=True)
    idxf = jnp.min(
        jnp.where(d == m, iotaf_ref[...], jnp.float32(N_CODES)), axis=1)
    idx_ref[0, 0, :] = idxf.astype(jnp.int32)
    mind = m[:, 0]

    @pl.when(i == 0)
    def _():
        loss_ref[...] = jnp.zeros((1, 1), jnp.float32)

    loss_ref[...] += jnp.sum(mind).reshape(1, 1)


def _dist_argmin(x2, W, wn):
    n_tiles = x2.shape[0] // TILE_R
    return pl.pallas_call(
        _dist_argmin_body,
        grid=(n_tiles,),
        in_specs=[
            pl.BlockSpec((TILE_R, E_DIM), lambda i: (i, 0)),
            pl.BlockSpec((N_CODES, E_DIM), lambda i: (0, 0)),
            pl.BlockSpec((1, N_CODES), lambda i: (0, 0)),
        ],
        out_specs=[
            pl.BlockSpec((1, 1, TILE_R), lambda i: (i, 0, 0)),
            pl.BlockSpec((1, 1), lambda i: (0, 0)),
        ],
        out_shape=[
            jax.ShapeDtypeStruct((n_tiles, 1, TILE_R), jnp.int32),
            jax.ShapeDtypeStruct((1, 1), jnp.float32),
        ],
        scratch_shapes=[
            pltpu.VMEM((1, N_CODES), jnp.float32),
        ],
    )(x2, W, wn)


def _make_sc_gather(n_rows):
    info = plsc.get_sparse_core_info()
    nc, ns = info.num_cores, info.num_subcores
    nw = nc * ns
    b_per_w = n_rows // nw
    n_chunks = b_per_w // IDX_CHUNK
    mesh = plsc.VectorSubcoreMesh(core_axis_name="c", subcore_axis_name="s")

    @functools.partial(
        pl.kernel,
        mesh=mesh,
        out_type=jax.ShapeDtypeStruct((n_rows, E_DIM), jnp.float32),
        scratch_types=[
            pltpu.VMEM((n_chunks, IDX_CHUNK), jnp.int32),
            pltpu.VMEM((b_per_w, E_DIM), jnp.float32),
            pltpu.SemaphoreType.DMA,
        ],
    )
    def gather(table_hbm, idx_hbm, out_hbm, idx_v, rows_v, sem):
        wid = lax.axis_index("s") * nc + lax.axis_index("c")
        pltpu.sync_copy(idx_hbm.at[pl.ds(wid * n_chunks, n_chunks)], idx_v)
        for j in range(n_chunks):
            pltpu.async_copy(
                table_hbm.at[idx_v.at[j]],
                rows_v.at[pl.ds(j * IDX_CHUNK, IDX_CHUNK)],
                sem,
            ).wait()
        pltpu.sync_copy(rows_v, out_hbm.at[pl.ds(wid * b_per_w, b_per_w)])

    return gather


def kernel(inputs, W):
    x2 = jnp.transpose(inputs, (0, 2, 3, 4, 1)).reshape(-1, E_DIM)
    wn = jnp.sum(W ** 2, axis=1).reshape(1, N_CODES)
    idx3, loss_acc = _dist_argmin(x2, W, wn)
    idx2d = idx3.reshape(N_ROWS // IDX_CHUNK, IDX_CHUNK)
    q = _make_sc_gather(N_ROWS)(W, idx2d)
    loss = loss_acc[0, 0] * (1.25 / (N_ROWS * E_DIM))
    out = jnp.transpose(q.reshape(2, 16, 16, 16, E_DIM), (0, 4, 1, 2, 3))
    return (loss, out)


# speedup vs baseline: 1.3364x; 1.0242x over previous
import functools

import jax
import jax.numpy as jnp
from jax import lax
from jax.experimental import pallas as pl
from jax.experimental.pallas import tpu as pltpu
from jax.experimental.pallas import tpu_sc as plsc

N_CODES = 8192
E_DIM = 256
N_ROWS = 8192
TILE_R = 1024
N_TILES = N_ROWS // TILE_R
IDX_CHUNK = 128


def _dist_argmin_body(x_ref, w_ref, idx_ref, loss_ref, wn_ref, iotaf_ref):
    i = pl.program_id(0)
    x = x_ref[...]
    w = w_ref[...]

    @pl.when(i == 0)
    def _():
        wn_col = jnp.sum(w * w, axis=1, keepdims=True)
        wn_ref[...] = lax.transpose(wn_col, (1, 0))
        iotaf_ref[...] = lax.broadcasted_iota(
            jnp.int32, (1, N_CODES), 1).astype(jnp.float32)

    mm = lax.dot_general(x, w, (((1,), (1,)), ((), ())),
                         preferred_element_type=jnp.float32)
    xn = jnp.sum(x * x, axis=1, keepdims=True)
    wn = wn_ref[...]
    d = (xn + wn) - 2.0 * mm
    m = jnp.min(d, axis=1, keepdims=True)
    idxf = jnp.min(
        jnp.where(d == m, iotaf_ref[...], jnp.float32(N_CODES)), axis=1)
    idx_ref[0, 0, :] = idxf.astype(jnp.int32)
    mind = m[:, 0]

    @pl.when(i == 0)
    def _():
        loss_ref[...] = jnp.zeros((1, 1), jnp.float32)

    loss_ref[...] += jnp.sum(mind).reshape(1, 1)


def _dist_argmin(x2, W):
    n_tiles = x2.shape[0] // TILE_R
    return pl.pallas_call(
        _dist_argmin_body,
        grid=(n_tiles,),
        in_specs=[
            pl.BlockSpec((TILE_R, E_DIM), lambda i: (i, 0)),
            pl.BlockSpec((N_CODES, E_DIM), lambda i: (0, 0)),
        ],
        out_specs=[
            pl.BlockSpec((1, 1, TILE_R), lambda i: (i, 0, 0)),
            pl.BlockSpec((1, 1), lambda i: (0, 0)),
        ],
        out_shape=[
            jax.ShapeDtypeStruct((n_tiles, 1, TILE_R), jnp.int32),
            jax.ShapeDtypeStruct((1, 1), jnp.float32),
        ],
        scratch_shapes=[
            pltpu.VMEM((1, N_CODES), jnp.float32),
            pltpu.VMEM((1, N_CODES), jnp.float32),
        ],
    )(x2, W)


def _make_sc_gather(n_rows):
    info = plsc.get_sparse_core_info()
    nc, ns = info.num_cores, info.num_subcores
    nw = nc * ns
    b_per_w = n_rows // nw
    n_chunks = b_per_w // IDX_CHUNK
    mesh = plsc.VectorSubcoreMesh(core_axis_name="c", subcore_axis_name="s")

    @functools.partial(
        pl.kernel,
        mesh=mesh,
        out_type=jax.ShapeDtypeStruct((n_rows, E_DIM), jnp.float32),
        scratch_types=[
            pltpu.VMEM((n_chunks, IDX_CHUNK), jnp.int32),
            pltpu.VMEM((b_per_w, E_DIM), jnp.float32),
            pltpu.SemaphoreType.DMA,
        ],
    )
    def gather(table_hbm, idx_hbm, out_hbm, idx_v, rows_v, sem):
        wid = lax.axis_index("s") * nc + lax.axis_index("c")
        pltpu.sync_copy(idx_hbm.at[pl.ds(wid * n_chunks, n_chunks)], idx_v)
        for j in range(n_chunks):
            pltpu.async_copy(
                table_hbm.at[idx_v.at[j]],
                rows_v.at[pl.ds(j * IDX_CHUNK, IDX_CHUNK)],
                sem,
            ).wait()
        pltpu.sync_copy(rows_v, out_hbm.at[pl.ds(wid * b_per_w, b_per_w)])

    return gather


def kernel(inputs, W):
    x2 = jnp.transpose(inputs, (0, 2, 3, 4, 1)).reshape(-1, E_DIM)
    idx3, loss_acc = _dist_argmin(x2, W)
    idx2d = idx3.reshape(N_ROWS // IDX_CHUNK, IDX_CHUNK)
    q = _make_sc_gather(N_ROWS)(W, idx2d)
    loss = loss_acc[0, 0] * (1.25 / (N_ROWS * E_DIM))
    out = jnp.transpose(q.reshape(2, 16, 16, 16, E_DIM), (0, 4, 1, 2, 3))
    return (loss, out)
